# baseline (speedup 1.0000x reference)
import functools

import jax
import jax.numpy as jnp
from jax import lax
from jax.experimental import pallas as pl
from jax.experimental.pallas import tpu as pltpu
from jax.experimental.pallas import tpu_sc as plsc

NWORDS = 1000000
NTAGS = 64
NTOK = 16384

NC = 2
NS = 16
LANES = 16
NW = NC * NS
B_PER_W = NTOK // NW
CHUNK = 128
NCHUNK = B_PER_W // CHUNK
NVEC = NTAGS // LANES

_mesh = plsc.VectorSubcoreMesh(
    core_axis_name="c", subcore_axis_name="s", num_cores=NC, num_subcores=NS
)


def _partial_sums_body(x_hbm, table_hbm, out_hbm, idx_v, rows_v, acc_v, sem):
    wid = lax.axis_index("s") * NC + lax.axis_index("c")
    base = wid * B_PER_W
    pltpu.sync_copy(x_hbm.at[pl.ds(base, B_PER_W)], idx_v)
    copies = []
    for j in range(NCHUNK):
        copies.append(
            pltpu.async_copy(
                table_hbm.at[idx_v.at[pl.ds(j * CHUNK, CHUNK)]],
                rows_v.at[pl.ds(j * CHUNK, CHUNK)],
                sem,
            )
        )
    for c in copies:
        c.wait()

    def body(i, acc):
        out = []
        for v in range(NVEC):
            out.append(acc[v] + rows_v[i, pl.ds(v * LANES, LANES)])
        return tuple(out)

    zero = jnp.zeros((LANES,), jnp.float32)
    acc = lax.fori_loop(0, B_PER_W, body, (zero,) * NVEC)
    for v in range(NVEC):
        acc_v[pl.ds(v * LANES, LANES)] = acc[v]
    pltpu.sync_copy(acc_v, out_hbm.at[wid])


_partial_sums = functools.partial(
    pl.kernel,
    mesh=_mesh,
    out_type=jax.ShapeDtypeStruct((NW, NTAGS), jnp.float32),
    scratch_types=[
        pltpu.VMEM((B_PER_W,), jnp.int32),
        pltpu.VMEM((B_PER_W, NTAGS), jnp.float32),
        pltpu.VMEM((NTAGS,), jnp.float32),
        pltpu.SemaphoreType.DMA,
    ],
    compiler_params=pltpu.CompilerParams(use_tc_tiling_on_sc=False),
)(_partial_sums_body)


def _finish_body(p_ref, b_ref, o_ref):
    o_ref[...] = jnp.sum(p_ref[...], axis=0, keepdims=True) + b_ref[...]


def kernel(x, table, bias):
    partials = _partial_sums(x.astype(jnp.int32), table)
    return pl.pallas_call(
        _finish_body,
        out_shape=jax.ShapeDtypeStruct((1, NTAGS), jnp.float32),
    )(partials, bias.reshape(1, NTAGS))


# speedup vs baseline: 4.7413x; 4.7413x over previous
import functools

import jax
import jax.numpy as jnp
from jax import lax
from jax.experimental import pallas as pl
from jax.experimental.pallas import tpu as pltpu
from jax.experimental.pallas import tpu_sc as plsc

NWORDS = 1000000
NTAGS = 64
NTOK = 16384

NC = 2
NS = 16
LANES = 16
B_PER_SC = NTOK // NC
B_PER_W = B_PER_SC // NS
SCHUNK = 128
NSCHUNK = B_PER_W // SCHUNK

W_PAD = 1 << 20
W_PER_TILE = W_PAD // NS
ZBUF = 8192

_mesh = plsc.VectorSubcoreMesh(
    core_axis_name="c", subcore_axis_name="s", num_cores=NC, num_subcores=NS
)


def _hist_body(x_hbm, out_hbm, idx_v, ones_v, zbuf_v, acc_sh):
    cid = lax.axis_index("c")
    sid = lax.axis_index("s")

    def zfill(i, carry):
        zbuf_v[pl.ds(i * LANES, LANES)] = jnp.zeros((LANES,), jnp.float32)
        return carry

    lax.fori_loop(0, ZBUF // LANES, zfill, 0)
    for k in range(W_PER_TILE // ZBUF):
        pltpu.sync_copy(
            zbuf_v, acc_sh.at[pl.ds(sid * W_PER_TILE + k * ZBUF, ZBUF)]
        )

    def ofill(i, carry):
        ones_v[0, pl.ds(i * LANES, LANES)] = jnp.ones((LANES,), jnp.float32)
        return carry

    lax.fori_loop(0, SCHUNK // LANES, ofill, 0)

    base = cid * B_PER_SC + sid * B_PER_W
    for j in range(NSCHUNK):
        pltpu.sync_copy(x_hbm.at[pl.ds(base + j * SCHUNK, SCHUNK)], idx_v.at[j])
    plsc.subcore_barrier()
    for j in range(NSCHUNK):
        pltpu.sync_copy(ones_v.at[0], acc_sh.at[idx_v.at[j]], add=True)
    plsc.subcore_barrier()
    pltpu.sync_copy(
        acc_sh.at[pl.ds(sid * W_PER_TILE, W_PER_TILE)],
        out_hbm.at[cid, pl.ds(sid * W_PER_TILE, W_PER_TILE)],
    )


_hist = functools.partial(
    pl.kernel,
    mesh=_mesh,
    out_type=jax.ShapeDtypeStruct((NC, W_PAD), jnp.float32),
    scratch_types=[
        pltpu.VMEM((NSCHUNK, SCHUNK), jnp.int32),
        pltpu.VMEM((1, SCHUNK), jnp.float32),
        pltpu.VMEM((ZBUF,), jnp.float32),
        pltpu.VMEM_SHARED((W_PAD,), jnp.float32),
    ],
)(_hist_body)

BW = 16384
_GRID = pl.cdiv(NWORDS, BW)


def _matvec_body(t_ref, c_ref, b_ref, o_ref, acc_ref):
    i = pl.program_id(0)

    @pl.when(i == 0)
    def _init():
        acc_ref[...] = jnp.zeros_like(acc_ref)

    col = i * BW + lax.broadcasted_iota(jnp.int32, (1, BW), 1)
    t = jnp.where(col < NWORDS, t_ref[...], 0.0)
    c = c_ref[0:1, :] + c_ref[1:2, :]
    acc_ref[...] += t * c

    @pl.when(i == _GRID - 1)
    def _fin():
        o_ref[...] = jnp.sum(acc_ref[...], axis=1)[None, :] + b_ref[...]


def kernel(x, table, bias):
    counts = _hist(x.astype(jnp.int32))
    table_t = table.T
    return pl.pallas_call(
        _matvec_body,
        grid=(_GRID,),
        in_specs=[
            pl.BlockSpec((NTAGS, BW), lambda i: (0, i)),
            pl.BlockSpec((NC, BW), lambda i: (0, i)),
            pl.BlockSpec((1, NTAGS), lambda i: (0, 0)),
        ],
        out_specs=pl.BlockSpec((1, NTAGS), lambda i: (0, 0)),
        out_shape=jax.ShapeDtypeStruct((1, NTAGS), jnp.float32),
        scratch_shapes=[pltpu.VMEM((NTAGS, BW), jnp.float32)],
        compiler_params=pltpu.CompilerParams(
            dimension_semantics=("arbitrary",)
        ),
    )(table_t, counts, bias.reshape(1, NTAGS))


# speedup vs baseline: 5.5184x; 1.1639x over previous
import functools

import jax
import jax.numpy as jnp
from jax import lax
from jax.experimental import pallas as pl
from jax.experimental.pallas import tpu as pltpu
from jax.experimental.pallas import tpu_sc as plsc

NWORDS = 1000000
NTAGS = 64
NTOK = 16384

NC = 2
NS = 16
LANES = 16
B_PER_SC = NTOK // NC
B_PER_W = B_PER_SC // NS
SCHUNK = 128
NSCHUNK = B_PER_W // SCHUNK

W_PAD = 1 << 20
W_PER_TILE = W_PAD // NS
ZBUF = 8192

_mesh = plsc.VectorSubcoreMesh(
    core_axis_name="c", subcore_axis_name="s", num_cores=NC, num_subcores=NS
)


def _hist_body(x_hbm, out_hbm, idx_v, ones_v, zbuf_v, acc_sh):
    cid = lax.axis_index("c")
    sid = lax.axis_index("s")

    def zfill(i, carry):
        zbuf_v[pl.ds(i * LANES, LANES)] = jnp.zeros((LANES,), jnp.float32)
        return carry

    lax.fori_loop(0, ZBUF // LANES, zfill, 0)
    for k in range(W_PER_TILE // ZBUF):
        pltpu.sync_copy(
            zbuf_v, acc_sh.at[pl.ds(sid * W_PER_TILE + k * ZBUF, ZBUF)]
        )

    def ofill(i, carry):
        ones_v[0, pl.ds(i * LANES, LANES)] = jnp.ones((LANES,), jnp.float32)
        return carry

    lax.fori_loop(0, SCHUNK // LANES, ofill, 0)

    base = cid * B_PER_SC + sid * B_PER_W
    for j in range(NSCHUNK):
        pltpu.sync_copy(x_hbm.at[pl.ds(base + j * SCHUNK, SCHUNK)], idx_v.at[j])
    plsc.subcore_barrier()
    for j in range(NSCHUNK):
        pltpu.sync_copy(ones_v.at[0], acc_sh.at[idx_v.at[j]], add=True)
    plsc.subcore_barrier()
    pltpu.sync_copy(
        acc_sh.at[pl.ds(sid * W_PER_TILE, W_PER_TILE)],
        out_hbm.at[cid, pl.ds(sid * W_PER_TILE, W_PER_TILE)],
    )


_hist = functools.partial(
    pl.kernel,
    mesh=_mesh,
    out_type=jax.ShapeDtypeStruct((NC, W_PAD), jnp.float32),
    scratch_types=[
        pltpu.VMEM((NSCHUNK, SCHUNK), jnp.int32),
        pltpu.VMEM((1, SCHUNK), jnp.float32),
        pltpu.VMEM((ZBUF,), jnp.float32),
        pltpu.VMEM_SHARED((W_PAD,), jnp.float32),
    ],
)(_hist_body)

BW = 32768
_GRID = pl.cdiv(NWORDS, BW)


def _matvec_body(t_ref, c_ref, b_ref, o_ref, acc_ref):
    i = pl.program_id(0)

    @pl.when(i == 0)
    def _init():
        acc_ref[...] = jnp.zeros_like(acc_ref)

    c = c_ref[0:1, :] + c_ref[1:2, :]
    acc_ref[...] += t_ref[...] * c

    @pl.when(i == _GRID - 1)
    def _fin():
        o_ref[...] = jnp.sum(acc_ref[...], axis=1)[None, :] + b_ref[...]


def kernel(x, table, bias):
    counts = _hist(x.astype(jnp.int32))
    table_t = table.T
    return pl.pallas_call(
        _matvec_body,
        grid=(_GRID,),
        in_specs=[
            pl.BlockSpec((NTAGS, BW), lambda i: (0, i)),
            pl.BlockSpec((NC, BW), lambda i: (0, i)),
            pl.BlockSpec((1, NTAGS), lambda i: (0, 0)),
        ],
        out_specs=pl.BlockSpec((1, NTAGS), lambda i: (0, 0)),
        out_shape=jax.ShapeDtypeStruct((1, NTAGS), jnp.float32),
        scratch_shapes=[pltpu.VMEM((NTAGS, BW), jnp.float32)],
        compiler_params=pltpu.CompilerParams(
            dimension_semantics=("arbitrary",)
        ),
    )(table_t, counts, bias.reshape(1, NTAGS))
